# baseline (speedup 1.0000x reference)
import functools

import jax
import jax.numpy as jnp
from jax import lax
from jax.experimental import pallas as pl
from jax.experimental.pallas import tpu as pltpu
from jax.experimental.pallas import tpu_sc as plsc

_MAX_NORM = 1.0
_EPS = 1e-7
_CW = 256
_CS = 8
_FL = 256


def _rsqrt_nr(x):
    i = plsc.bitcast(x, jnp.int32)
    i = jnp.int32(0x5F3759DF) - (i >> 1)
    y = plsc.bitcast(i, jnp.float32)
    for _ in range(3):
        y = y * (1.5 - 0.5 * x * y * y)
    return y


def _band_bounds(wid, CPB, V, NW):
    lo = wid * (CPB * _CW)
    hi = jnp.where(wid == NW - 1, jnp.int32(V), lo + CPB * _CW)
    return lo, hi


@functools.cache
def _build_count(NW, NC, B, V, CPB):
    mesh = plsc.VectorSubcoreMesh(core_axis_name="c", subcore_axis_name="s")
    cnt_ty = jax.ShapeDtypeStruct((8 * NW,), jnp.int32)

    @functools.partial(
        pl.kernel,
        mesh=mesh,
        out_type=(cnt_ty, cnt_ty),
        scratch_types=[
            pltpu.VMEM((B,), jnp.int32),
            pltpu.VMEM((16,), jnp.int32),
        ],
        compiler_params=pltpu.CompilerParams(needs_layout_passes=False),
    )
    def k(uids_hbm, iids_hbm, ucnt_hbm, icnt_hbm, idsb, cvm):
        wid = lax.axis_index("s") * NC + lax.axis_index("c")
        lo, hi = _band_bounds(wid, CPB, V, NW)

        for ids_hbm, cnt_hbm in ((uids_hbm, ucnt_hbm), (iids_hbm, icnt_hbm)):
            pltpu.sync_copy(ids_hbm, idsb)

            def inner(j, t):
                idv = idsb[pl.ds(j * 16, 16)]
                m = (idv >= lo) & (idv < hi)
                return t + plsc.all_reduce_population_count(m)

            tot = lax.fori_loop(0, B // 16, inner,
                                jnp.zeros((16,), jnp.int32))
            cvm[pl.ds(0, 16)] = tot
            pltpu.sync_copy(cvm.at[pl.ds(0, 8)],
                            cnt_hbm.at[pl.ds(wid * 8, 8)])

    return k


@functools.cache
def _build_extract(NW, NC, D, B, V, CPB, NCHK, EXTP):
    W = 2 * D
    TW = V - (NCHK - 1) * _CW
    mesh = plsc.VectorSubcoreMesh(core_axis_name="c", subcore_axis_name="s")
    ext_ty = jax.ShapeDtypeStruct((EXTP // 2, W), jnp.float32)
    bor_ty = jax.ShapeDtypeStruct((EXTP,), jnp.int32)

    @functools.partial(
        pl.kernel,
        mesh=mesh,
        out_type=(ext_ty, ext_ty, bor_ty, bor_ty),
        scratch_types=[
            pltpu.VMEM((B + 16,), jnp.int32),
            pltpu.VMEM((B + 16,), jnp.int32),
            pltpu.VMEM((4096,), jnp.int32),
            pltpu.VMEM((8 * NW + 16,), jnp.int32),
            pltpu.VMEM((D, _CW), jnp.float32),
            pltpu.VMEM((D, _CW), jnp.float32),
            pltpu.VMEM((D, _CW), jnp.float32),
            pltpu.VMEM((D, _CW), jnp.float32),
            pltpu.VMEM((D, TW), jnp.float32),
            pltpu.VMEM((_FL // 2, W), jnp.float32),
            pltpu.VMEM((_FL + 16,), jnp.int32),
            pltpu.VMEM((32,), jnp.int32),
            pltpu.VMEM((32,), jnp.int32),
            pltpu.SemaphoreType.DMA,
            pltpu.SemaphoreType.DMA,
            pltpu.SemaphoreType.DMA,
            pltpu.SemaphoreType.DMA,
        ],
        compiler_params=pltpu.CompilerParams(needs_layout_passes=False),
    )
    def k(ut_hbm, it_hbm, utail_hbm, itail_hbm, uids_hbm, iids_hbm,
          ucnt_hbm, icnt_hbm, uext_hbm, vext_hbm, ubor_hbm, vbor_hbm,
          wlid, wlb, idsb, cnts, winA, winB, winC, winD, wtail, stage,
          borst, mids, mbs, semA, semB, semC, semD):
        wid = lax.axis_index("s") * NC + lax.axis_index("c")
        lo, hi = _band_bounds(wid, CPB, V, NW)
        lanes = lax.iota(jnp.int32, 16)
        senti = jnp.full((16,), B, dtype=jnp.int32)

        def sread(ref, i):
            return ref[pl.ds(i, 16)][0]

        for (tab_hbm, tail_hbm, ids_hbm, cnt_hbm, ext_hbm, bor_hbm) in (
                (ut_hbm, utail_hbm, uids_hbm, ucnt_hbm, uext_hbm, ubor_hbm),
                (it_hbm, itail_hbm, iids_hbm, icnt_hbm, vext_hbm, vbor_hbm)):
            pltpu.sync_copy(cnt_hbm, cnts.at[pl.ds(0, 8 * NW)])
            pltpu.sync_copy(tail_hbm, wtail)

            def pref(j, acc):
                cj = sread(cnts, j * 8)
                return acc + ((cj + _FL - 1) >> 8 << 8)

            base = lax.fori_loop(0, wid, pref, jnp.int32(0))

            def binit(kk, _):
                borst[pl.ds(kk * 16, 16)] = senti
                return 0

            lax.fori_loop(0, _FL // 16, binit, 0)

            def blk(kk, ptr):
                pltpu.sync_copy(ids_hbm.at[pl.ds(kk * 4096, 4096)], idsb)

                def inner(j, p):
                    idv = idsb[pl.ds(j * 16, 16)]
                    bv = kk * 4096 + j * 16 + lanes
                    m = (idv >= lo) & (idv < hi)
                    plsc.store_compressed(wlid.at[pl.ds(p, 16)], idv, mask=m)
                    plsc.store_compressed(wlb.at[pl.ds(p, 16)], bv, mask=m)
                    return p + plsc.all_reduce_population_count(m)[0]

                return lax.fori_loop(0, 4096 // 16, inner, ptr)

            nw = lax.fori_loop(0, B // 4096, blk, jnp.int32(0))
            nblk = (nw + 15) >> 4

            nch = jnp.where(wid == NW - 1,
                            jnp.int32(NCHK - (NW - 1) * CPB),
                            jnp.int32(CPB))
            nch_main = nch - jnp.where(wid == NW - 1, 1, 0)

            def scan_chunk(cglob, buf, carry):
                def scan_blk(kk, carry2):
                    slot2, bacc2 = carry2
                    wlv = wlid[pl.ds(kk * 16, 16)]
                    wbv = wlb[pl.ds(kk * 16, 16)]
                    valid = (kk * 16 + lanes) < nw
                    m = ((wlv >> _CS) == cglob) & valid
                    plsc.store_compressed(mids.at[pl.ds(0, 16)], wlv, mask=m)
                    plsc.store_compressed(mbs.at[pl.ds(0, 16)], wbv, mask=m)
                    mc = plsc.all_reduce_population_count(m)[0]

                    def emit(j, carry3):
                        slot3, bacc3 = carry3
                        lcol = sread(mids, j) - cglob * _CW
                        bval = sread(mbs, j)
                        cols = jnp.full((16,), lcol, dtype=jnp.int32)
                        sl = slot3 & (_FL - 1)
                        cbase = (sl & 1) * D
                        for o in range(D // 16):
                            x16 = plsc.load_gather(
                                buf, [o * 16 + lanes, cols])
                            stage[sl >> 1, pl.ds(cbase + o * 16, 16)] = x16
                        bacc3 = jnp.where(lanes == (sl & 15),
                                          jnp.full((16,), bval, jnp.int32),
                                          bacc3)

                        @pl.when((sl & 15) == 15)
                        def _():
                            borst[pl.ds(sl & ~15, 16)] = bacc3

                        bacc3 = jnp.where((sl & 15) == 15, senti, bacc3)

                        @pl.when(sl == _FL - 1)
                        def _():
                            off = pl.multiple_of(
                                base + (slot3 & ~(_FL - 1)), _FL)
                            pr = pl.multiple_of(off >> 1, _FL // 2)
                            pltpu.sync_copy(
                                stage, ext_hbm.at[pl.ds(pr, _FL // 2)])
                            pltpu.sync_copy(
                                borst.at[pl.ds(0, _FL)],
                                bor_hbm.at[pl.ds(off, _FL)])

                            def breset(kk, _):
                                borst[pl.ds(kk * 16, 16)] = senti
                                return 0

                            lax.fori_loop(0, _FL // 16, breset, 0)

                        return slot3 + 1, bacc3

                    return lax.fori_loop(0, mc, emit, (slot2, bacc2))

                return lax.fori_loop(0, nblk, scan_blk, carry)

            wins = (winA, winB, winC, winD)
            wsems = (semA, semB, semC, semD)

            def fire_win(crel, buf, sem):
                cg = wid * CPB + crel
                col0 = pl.multiple_of(cg * _CW, 128)
                pltpu.async_copy(tab_hbm.at[:, pl.ds(col0, _CW)], buf, sem)

            def do_chunk(crel, carry, r):
                buf, sem = wins[r], wsems[r]
                pltpu.make_async_copy(
                    tab_hbm.at[:, pl.ds(0, _CW)], buf, sem).wait()

                @pl.when(crel + 3 < nch_main)
                def _():
                    fire_win(crel + 3, wins[(r + 3) % 4], wsems[(r + 3) % 4])

                return scan_chunk(wid * CPB + crel, buf, carry)

            for r in range(3):
                @pl.when(r < nch_main)
                def _(r=r):
                    fire_win(r, wins[r], wsems[r])

            def chunk_body(crel, carry):
                r = crel & 3
                return lax.cond(
                    r < 2,
                    lambda c: lax.cond(
                        r == 0,
                        lambda c2: do_chunk(crel, c2, 0),
                        lambda c2: do_chunk(crel, c2, 1),
                        c),
                    lambda c: lax.cond(
                        r == 2,
                        lambda c2: do_chunk(crel, c2, 2),
                        lambda c2: do_chunk(crel, c2, 3),
                        c),
                    carry)

            carry = lax.fori_loop(
                0, nch_main, chunk_body, (jnp.int32(0), senti))
            slot, bacc = lax.cond(
                wid == NW - 1,
                lambda c: scan_chunk(jnp.int32(NCHK - 1), wtail, c),
                lambda c: c,
                carry)

            rem = slot & (_FL - 1)

            @pl.when(rem > 0)
            def _():
                @pl.when((rem & 15) > 0)
                def _():
                    borst[pl.ds(rem & ~15, 16)] = bacc

                off = pl.multiple_of(base + (slot & ~(_FL - 1)), _FL)
                pr = pl.multiple_of(off >> 1, _FL // 2)
                pltpu.sync_copy(stage, ext_hbm.at[pl.ds(pr, _FL // 2)])
                pltpu.sync_copy(borst.at[pl.ds(0, _FL)],
                                bor_hbm.at[pl.ds(off, _FL)])

    return k


@functools.cache
def _build_join(NW, NC, D, B, EXTP):
    bpw = B // NW
    W = 2 * D
    C = 128
    NCH = bpw // C
    mesh = plsc.VectorSubcoreMesh(core_axis_name="c", subcore_axis_name="s")

    @functools.partial(
        pl.kernel,
        mesh=mesh,
        out_type=jax.ShapeDtypeStruct((B,), jnp.float32),
        scratch_types=[
            pltpu.VMEM((B,), jnp.int32),
            pltpu.VMEM((B,), jnp.int32),
            pltpu.VMEM((4096,), jnp.int32),
            pltpu.VMEM((8 * NW + 16,), jnp.int32),
            pltpu.VMEM((bpw,), jnp.int32),
            pltpu.VMEM((bpw,), jnp.int32),
            pltpu.VMEM((bpw,), jnp.int32),
            pltpu.VMEM((bpw,), jnp.int32),
            pltpu.VMEM((C, W), jnp.float32),
            pltpu.VMEM((C, W), jnp.float32),
            pltpu.VMEM((C, W), jnp.float32),
            pltpu.VMEM((C, W), jnp.float32),
            pltpu.VMEM((bpw,), jnp.float32),
            pltpu.SemaphoreType.DMA,
            pltpu.SemaphoreType.DMA,
        ],
        compiler_params=pltpu.CompilerParams(needs_layout_passes=False),
    )
    def k(uext_hbm, vext_hbm, ubor_hbm, vbor_hbm, ucnt_hbm, icnt_hbm,
          out_hbm, posu, posi, borb, cnts, uidx, iidx, ucol, icol,
          ubufa, ubufb, ibufa, ibufb, obuf, sema, semb):
        wid = lax.axis_index("s") * NC + lax.axis_index("c")
        lanes = lax.iota(jnp.int32, 16)
        p0 = wid * bpw

        def sread(ref, i):
            return ref[pl.ds(i, 16)][0]

        for bor_hbm, cnt_hbm, pos in ((ubor_hbm, ucnt_hbm, posu),
                                      (vbor_hbm, icnt_hbm, posi)):
            pltpu.sync_copy(cnt_hbm, cnts.at[pl.ds(0, 8 * NW)])

            def pref(j, acc):
                cj = sread(cnts, j * 8)
                return acc + ((cj + _FL - 1) >> 8 << 8)

            tot = lax.fori_loop(0, NW, pref, jnp.int32(0))

            def scat(blk, _):
                pltpu.sync_copy(bor_hbm.at[pl.ds(blk * 4096, 4096)], borb)

                def inner(kk, _):
                    bv = borb[pl.ds(kk * 16, 16)]
                    sv = blk * 4096 + kk * 16 + lanes
                    m = (bv < B) & (sv < tot)
                    plsc.store_scatter(pos, [bv], sv, mask=m)
                    return 0

                return lax.fori_loop(0, 4096 // 16, inner, 0)

            lax.fori_loop(0, EXTP // 4096, scat, 0)

        def mkidx(kk, _):
            pu = posu[pl.ds(p0 + kk * 16, 16)]
            pv = posi[pl.ds(p0 + kk * 16, 16)]
            uidx[pl.ds(kk * 16, 16)] = pu >> 1
            iidx[pl.ds(kk * 16, 16)] = pv >> 1
            ucol[pl.ds(kk * 16, 16)] = (pu & 1) * D
            icol[pl.ds(kk * 16, 16)] = (pv & 1) * D
            return 0

        lax.fori_loop(0, bpw // 16, mkidx, 0)

        ubufs, ibufs, sems = [ubufa, ubufb], [ibufa, ibufb], [sema, semb]

        def fire(j):
            p = j % 2
            return [
                pltpu.async_copy(uext_hbm.at[uidx.at[pl.ds(j * C, C)]],
                                 ubufs[p], sems[p]),
                pltpu.async_copy(vext_hbm.at[iidx.at[pl.ds(j * C, C)]],
                                 ibufs[p], sems[p]),
            ]

        zeros = jnp.zeros((16,), jnp.float32)
        eps2 = jnp.float32(_EPS * _EPS)

        pending = fire(0)
        for j in range(NCH):
            p = j % 2
            for cp in pending:
                cp.wait()
            if j + 1 < NCH:
                pending = fire(j + 1)
            ubuf, ibuf = ubufs[p], ibufs[p]

            def group_body(g, _, j=j, ubuf=ubuf, ibuf=ibuf):
                rows = g * 16 + lanes
                ucb = ucol[pl.ds(j * C + g * 16, 16)]
                icb = icol[pl.ds(j * C + g * 16, 16)]

                def d_body(d, carry):
                    acc, nu, nv = carry
                    u = plsc.load_gather(ubuf, [rows, ucb + d])
                    v = plsc.load_gather(ibuf, [rows, icb + d])
                    return acc + u * v, nu + u * u, nv + v * v

                acc, nu, nv = lax.fori_loop(
                    0, D, d_body, (zeros, zeros, zeros))
                su = jnp.minimum(jnp.float32(_MAX_NORM),
                                 _rsqrt_nr(jnp.maximum(nu, eps2)))
                sv = jnp.minimum(jnp.float32(_MAX_NORM),
                                 _rsqrt_nr(jnp.maximum(nv, eps2)))
                x = acc * su * sv
                obuf[pl.ds(j * C + g * 16, 16)] = 1.0 / (1.0 + jnp.exp(-x))
                return 0

            lax.fori_loop(0, C // 16, group_body, 0)

        pltpu.sync_copy(obuf, out_hbm.at[pl.ds(p0, bpw)])

    return k


@jax.jit
def kernel(user_ids, item_ids, users, items):
    B = user_ids.shape[0]
    V, D = users.shape
    info = plsc.get_sparse_core_info()
    NC, NS = info.num_cores, info.num_subcores
    NW = NC * NS
    NCHK = -(-V // _CW)
    CPB = NCHK // NW
    EXTP = B + _FL * NW

    uid = user_ids.astype(jnp.int32)
    iid = item_ids.astype(jnp.int32)
    last_w = (NCHK - 1) * _CW

    kc = _build_count(NW, NC, B, V, CPB)
    ucnt, icnt = kc(uid, iid)

    kx = _build_extract(NW, NC, D, B, V, CPB, NCHK, EXTP)
    uext, vext, ubor, vbor = kx(
        users.T, items.T, users.T[:, last_w:], items.T[:, last_w:],
        uid, iid, ucnt, icnt)

    kj = _build_join(NW, NC, D, B, EXTP)
    return kj(uext, vext, ubor, vbor, ucnt, icnt)


# speedup vs baseline: 1.1433x; 1.1433x over previous
import functools

import jax
import jax.numpy as jnp
from jax import lax
from jax.experimental import pallas as pl
from jax.experimental.pallas import tpu as pltpu
from jax.experimental.pallas import tpu_sc as plsc

_MAX_NORM = 1.0
_EPS = 1e-7
_CW = 512
_FL = 256


def _rsqrt_nr(x):
    i = plsc.bitcast(x, jnp.int32)
    i = jnp.int32(0x5F3759DF) - (i >> 1)
    y = plsc.bitcast(i, jnp.float32)
    for _ in range(3):
        y = y * (1.5 - 0.5 * x * y * y)
    return y


def _band_bounds(wid, CPB, V, NW):
    lo = wid * (CPB * _CW)
    hi = jnp.where(wid == NW - 1, jnp.int32(V), lo + CPB * _CW)
    return lo, hi


@functools.cache
def _build_count(NW, NC, B, V, CPB):
    mesh = plsc.VectorSubcoreMesh(core_axis_name="c", subcore_axis_name="s")
    cnt_ty = jax.ShapeDtypeStruct((8 * NW,), jnp.int32)

    @functools.partial(
        pl.kernel,
        mesh=mesh,
        out_type=(cnt_ty, cnt_ty),
        scratch_types=[
            pltpu.VMEM((B,), jnp.int32),
            pltpu.VMEM((16,), jnp.int32),
        ],
        compiler_params=pltpu.CompilerParams(needs_layout_passes=False),
    )
    def k(uids_hbm, iids_hbm, ucnt_hbm, icnt_hbm, idsb, cvm):
        wid = lax.axis_index("s") * NC + lax.axis_index("c")
        lo, hi = _band_bounds(wid, CPB, V, NW)

        for ids_hbm, cnt_hbm in ((uids_hbm, ucnt_hbm), (iids_hbm, icnt_hbm)):
            pltpu.sync_copy(ids_hbm, idsb)

            def inner(j, t):
                idv = idsb[pl.ds(j * 16, 16)]
                m = (idv >= lo) & (idv < hi)
                return t + plsc.all_reduce_population_count(m)

            tot = lax.fori_loop(0, B // 16, inner,
                                jnp.zeros((16,), jnp.int32))
            cvm[pl.ds(0, 16)] = tot
            pltpu.sync_copy(cvm.at[pl.ds(0, 8)],
                            cnt_hbm.at[pl.ds(wid * 8, 8)])

    return k


@functools.cache
def _build_extract(NW, NC, D, B, V, CPB, NCHK, EXTP):
    W = 2 * D
    TW = V - (NCHK - 1) * _CW
    mesh = plsc.VectorSubcoreMesh(core_axis_name="c", subcore_axis_name="s")
    ext_ty = jax.ShapeDtypeStruct((EXTP // 2, W), jnp.float32)
    bor_ty = jax.ShapeDtypeStruct((EXTP,), jnp.int32)

    @functools.partial(
        pl.kernel,
        mesh=mesh,
        out_type=(ext_ty, ext_ty, bor_ty, bor_ty),
        scratch_types=[
            pltpu.VMEM((B + 16,), jnp.int32),
            pltpu.VMEM((B + 16,), jnp.int32),
            pltpu.VMEM((4096,), jnp.int32),
            pltpu.VMEM((8 * NW + 16,), jnp.int32),
            pltpu.VMEM((D, _CW), jnp.float32),
            pltpu.VMEM((D, _CW), jnp.float32),
            pltpu.VMEM((D, TW), jnp.float32),
            pltpu.VMEM((_FL // 2, W), jnp.float32),
            pltpu.VMEM((_FL + 16,), jnp.int32),
            pltpu.VMEM((32,), jnp.int32),
            pltpu.VMEM((32,), jnp.int32),
            pltpu.SemaphoreType.DMA,
            pltpu.SemaphoreType.DMA,
        ],
        compiler_params=pltpu.CompilerParams(needs_layout_passes=False),
    )
    def k(ut_hbm, it_hbm, utail_hbm, itail_hbm, uids_hbm, iids_hbm,
          ucnt_hbm, icnt_hbm, uext_hbm, vext_hbm, ubor_hbm, vbor_hbm,
          wlid, wlb, idsb, cnts, winA, winB, wtail, stage, borst,
          mids, mbs, semA, semB):
        wid = lax.axis_index("s") * NC + lax.axis_index("c")
        lo, hi = _band_bounds(wid, CPB, V, NW)
        lanes = lax.iota(jnp.int32, 16)
        senti = jnp.full((16,), B, dtype=jnp.int32)

        def sread(ref, i):
            return ref[pl.ds(i, 16)][0]

        for (tab_hbm, tail_hbm, ids_hbm, cnt_hbm, ext_hbm, bor_hbm) in (
                (ut_hbm, utail_hbm, uids_hbm, ucnt_hbm, uext_hbm, ubor_hbm),
                (it_hbm, itail_hbm, iids_hbm, icnt_hbm, vext_hbm, vbor_hbm)):
            pltpu.sync_copy(cnt_hbm, cnts.at[pl.ds(0, 8 * NW)])
            pltpu.sync_copy(tail_hbm, wtail)

            def pref(j, acc):
                cj = sread(cnts, j * 8)
                return acc + ((cj + _FL - 1) >> 8 << 8)

            base = lax.fori_loop(0, wid, pref, jnp.int32(0))

            def binit(kk, _):
                borst[pl.ds(kk * 16, 16)] = senti
                return 0

            lax.fori_loop(0, _FL // 16, binit, 0)

            def blk(kk, ptr):
                pltpu.sync_copy(ids_hbm.at[pl.ds(kk * 4096, 4096)], idsb)

                def inner(j, p):
                    idv = idsb[pl.ds(j * 16, 16)]
                    bv = kk * 4096 + j * 16 + lanes
                    m = (idv >= lo) & (idv < hi)
                    plsc.store_compressed(wlid.at[pl.ds(p, 16)], idv, mask=m)
                    plsc.store_compressed(wlb.at[pl.ds(p, 16)], bv, mask=m)
                    return p + plsc.all_reduce_population_count(m)[0]

                return lax.fori_loop(0, 4096 // 16, inner, ptr)

            nw = lax.fori_loop(0, B // 4096, blk, jnp.int32(0))
            nblk = (nw + 15) >> 4

            nch = jnp.where(wid == NW - 1,
                            jnp.int32(NCHK - (NW - 1) * CPB),
                            jnp.int32(CPB))
            nch_main = nch - jnp.where(wid == NW - 1, 1, 0)

            def scan_chunk(cglob, buf, carry):
                def scan_blk(kk, carry2):
                    slot2, bacc2 = carry2
                    wlv = wlid[pl.ds(kk * 16, 16)]
                    wbv = wlb[pl.ds(kk * 16, 16)]
                    valid = (kk * 16 + lanes) < nw
                    m = ((wlv >> 9) == cglob) & valid
                    plsc.store_compressed(mids.at[pl.ds(0, 16)], wlv, mask=m)
                    plsc.store_compressed(mbs.at[pl.ds(0, 16)], wbv, mask=m)
                    mc = plsc.all_reduce_population_count(m)[0]

                    def emit(j, carry3):
                        slot3, bacc3 = carry3
                        lcol = sread(mids, j) - cglob * _CW
                        bval = sread(mbs, j)
                        cols = jnp.full((16,), lcol, dtype=jnp.int32)
                        sl = slot3 & (_FL - 1)
                        cbase = (sl & 1) * D
                        for o in range(D // 16):
                            x16 = plsc.load_gather(
                                buf, [o * 16 + lanes, cols])
                            stage[sl >> 1, pl.ds(cbase + o * 16, 16)] = x16
                        bacc3 = jnp.where(lanes == (sl & 15),
                                          jnp.full((16,), bval, jnp.int32),
                                          bacc3)

                        @pl.when((sl & 15) == 15)
                        def _():
                            borst[pl.ds(sl & ~15, 16)] = bacc3

                        bacc3 = jnp.where((sl & 15) == 15, senti, bacc3)

                        @pl.when(sl == _FL - 1)
                        def _():
                            off = pl.multiple_of(
                                base + (slot3 & ~(_FL - 1)), _FL)
                            pr = pl.multiple_of(off >> 1, _FL // 2)
                            pltpu.sync_copy(
                                stage, ext_hbm.at[pl.ds(pr, _FL // 2)])
                            pltpu.sync_copy(
                                borst.at[pl.ds(0, _FL)],
                                bor_hbm.at[pl.ds(off, _FL)])

                            def breset(kk, _):
                                borst[pl.ds(kk * 16, 16)] = senti
                                return 0

                            lax.fori_loop(0, _FL // 16, breset, 0)

                        return slot3 + 1, bacc3

                    return lax.fori_loop(0, mc, emit, (slot2, bacc2))

                return lax.fori_loop(0, nblk, scan_blk, carry)

            def fire_win(crel, buf, sem):
                cg = wid * CPB + crel
                col0 = pl.multiple_of(cg * _CW, 128)
                pltpu.async_copy(tab_hbm.at[:, pl.ds(col0, _CW)], buf, sem)

            def do_chunk(crel, carry, buf, sem, obuf, osem):
                pltpu.make_async_copy(
                    tab_hbm.at[:, pl.ds(0, _CW)], buf, sem).wait()

                @pl.when(crel + 1 < nch_main)
                def _():
                    fire_win(crel + 1, obuf, osem)

                return scan_chunk(wid * CPB + crel, buf, carry)

            @pl.when(nch_main > 0)
            def _():
                fire_win(0, winA, semA)

            def chunk_body(crel, carry):
                return lax.cond(
                    (crel & 1) == 0,
                    lambda c: do_chunk(crel, c, winA, semA, winB, semB),
                    lambda c: do_chunk(crel, c, winB, semB, winA, semA),
                    carry)

            carry = lax.fori_loop(
                0, nch_main, chunk_body, (jnp.int32(0), senti))
            slot, bacc = lax.cond(
                wid == NW - 1,
                lambda c: scan_chunk(jnp.int32(NCHK - 1), wtail, c),
                lambda c: c,
                carry)

            rem = slot & (_FL - 1)

            @pl.when(rem > 0)
            def _():
                @pl.when((rem & 15) > 0)
                def _():
                    borst[pl.ds(rem & ~15, 16)] = bacc

                off = pl.multiple_of(base + (slot & ~(_FL - 1)), _FL)
                pr = pl.multiple_of(off >> 1, _FL // 2)
                pltpu.sync_copy(stage, ext_hbm.at[pl.ds(pr, _FL // 2)])
                pltpu.sync_copy(borst.at[pl.ds(0, _FL)],
                                bor_hbm.at[pl.ds(off, _FL)])

    return k


@functools.cache
def _build_join(NW, NC, D, B, EXTP):
    bpw = B // NW
    W = 2 * D
    C = 128
    NCH = bpw // C
    mesh = plsc.VectorSubcoreMesh(core_axis_name="c", subcore_axis_name="s")

    @functools.partial(
        pl.kernel,
        mesh=mesh,
        out_type=jax.ShapeDtypeStruct((B,), jnp.float32),
        scratch_types=[
            pltpu.VMEM((B,), jnp.int32),
            pltpu.VMEM((B,), jnp.int32),
            pltpu.VMEM((4096,), jnp.int32),
            pltpu.VMEM((8 * NW + 16,), jnp.int32),
            pltpu.VMEM((bpw,), jnp.int32),
            pltpu.VMEM((bpw,), jnp.int32),
            pltpu.VMEM((bpw,), jnp.int32),
            pltpu.VMEM((bpw,), jnp.int32),
            pltpu.VMEM((C, W), jnp.float32),
            pltpu.VMEM((C, W), jnp.float32),
            pltpu.VMEM((C, W), jnp.float32),
            pltpu.VMEM((C, W), jnp.float32),
            pltpu.VMEM((bpw,), jnp.float32),
            pltpu.SemaphoreType.DMA,
            pltpu.SemaphoreType.DMA,
        ],
        compiler_params=pltpu.CompilerParams(needs_layout_passes=False),
    )
    def k(uext_hbm, vext_hbm, ubor_hbm, vbor_hbm, ucnt_hbm, icnt_hbm,
          out_hbm, posu, posi, borb, cnts, uidx, iidx, ucol, icol,
          ubufa, ubufb, ibufa, ibufb, obuf, sema, semb):
        wid = lax.axis_index("s") * NC + lax.axis_index("c")
        lanes = lax.iota(jnp.int32, 16)
        p0 = wid * bpw

        def sread(ref, i):
            return ref[pl.ds(i, 16)][0]

        for bor_hbm, cnt_hbm, pos in ((ubor_hbm, ucnt_hbm, posu),
                                      (vbor_hbm, icnt_hbm, posi)):
            pltpu.sync_copy(cnt_hbm, cnts.at[pl.ds(0, 8 * NW)])

            def pref(j, acc):
                cj = sread(cnts, j * 8)
                return acc + ((cj + _FL - 1) >> 8 << 8)

            tot = lax.fori_loop(0, NW, pref, jnp.int32(0))

            def scat(blk, _):
                pltpu.sync_copy(bor_hbm.at[pl.ds(blk * 4096, 4096)], borb)

                def inner(kk, _):
                    bv = borb[pl.ds(kk * 16, 16)]
                    sv = blk * 4096 + kk * 16 + lanes
                    m = (bv < B) & (sv < tot)
                    plsc.store_scatter(pos, [bv], sv, mask=m)
                    return 0

                return lax.fori_loop(0, 4096 // 16, inner, 0)

            lax.fori_loop(0, EXTP // 4096, scat, 0)

        def mkidx(kk, _):
            pu = posu[pl.ds(p0 + kk * 16, 16)]
            pv = posi[pl.ds(p0 + kk * 16, 16)]
            uidx[pl.ds(kk * 16, 16)] = pu >> 1
            iidx[pl.ds(kk * 16, 16)] = pv >> 1
            ucol[pl.ds(kk * 16, 16)] = (pu & 1) * D
            icol[pl.ds(kk * 16, 16)] = (pv & 1) * D
            return 0

        lax.fori_loop(0, bpw // 16, mkidx, 0)

        ubufs, ibufs, sems = [ubufa, ubufb], [ibufa, ibufb], [sema, semb]

        def fire(j):
            p = j % 2
            return [
                pltpu.async_copy(uext_hbm.at[uidx.at[pl.ds(j * C, C)]],
                                 ubufs[p], sems[p]),
                pltpu.async_copy(vext_hbm.at[iidx.at[pl.ds(j * C, C)]],
                                 ibufs[p], sems[p]),
            ]

        zeros = jnp.zeros((16,), jnp.float32)
        eps2 = jnp.float32(_EPS * _EPS)

        pending = fire(0)
        for j in range(NCH):
            p = j % 2
            for cp in pending:
                cp.wait()
            if j + 1 < NCH:
                pending = fire(j + 1)
            ubuf, ibuf = ubufs[p], ibufs[p]

            def group_body(g, _, j=j, ubuf=ubuf, ibuf=ibuf):
                rows = g * 16 + lanes
                ucb = ucol[pl.ds(j * C + g * 16, 16)]
                icb = icol[pl.ds(j * C + g * 16, 16)]

                def d_body(d, carry):
                    acc, nu, nv = carry
                    u = plsc.load_gather(ubuf, [rows, ucb + d])
                    v = plsc.load_gather(ibuf, [rows, icb + d])
                    return acc + u * v, nu + u * u, nv + v * v

                acc, nu, nv = lax.fori_loop(
                    0, D, d_body, (zeros, zeros, zeros))
                su = jnp.minimum(jnp.float32(_MAX_NORM),
                                 _rsqrt_nr(jnp.maximum(nu, eps2)))
                sv = jnp.minimum(jnp.float32(_MAX_NORM),
                                 _rsqrt_nr(jnp.maximum(nv, eps2)))
                x = acc * su * sv
                obuf[pl.ds(j * C + g * 16, 16)] = 1.0 / (1.0 + jnp.exp(-x))
                return 0

            lax.fori_loop(0, C // 16, group_body, 0)

        pltpu.sync_copy(obuf, out_hbm.at[pl.ds(p0, bpw)])

    return k


@jax.jit
def kernel(user_ids, item_ids, users, items):
    B = user_ids.shape[0]
    V, D = users.shape
    info = plsc.get_sparse_core_info()
    NC, NS = info.num_cores, info.num_subcores
    NW = NC * NS
    NCHK = -(-V // _CW)
    CPB = NCHK // NW
    EXTP = B + _FL * NW

    uid = user_ids.astype(jnp.int32)
    iid = item_ids.astype(jnp.int32)
    last_w = (NCHK - 1) * _CW

    kc = _build_count(NW, NC, B, V, CPB)
    ucnt, icnt = kc(uid, iid)

    kx = _build_extract(NW, NC, D, B, V, CPB, NCHK, EXTP)
    uext, vext, ubor, vbor = kx(
        users.T, items.T, users.T[:, last_w:], items.T[:, last_w:],
        uid, iid, ucnt, icnt)

    kj = _build_join(NW, NC, D, B, EXTP)
    return kj(uext, vext, ubor, vbor, ucnt, icnt)
